# Initial kernel scaffold; baseline (speedup 1.0000x reference)
import functools

import jax
import jax.numpy as jnp
from jax import lax
from jax.experimental import pallas as pl
from jax.experimental.pallas import tpu as pltpu
from jax.experimental.pallas import tpu_sc as plsc

N = 10000
NP = 10240
D = 128
NC, NS, L = 2, 16, 16
NW = NC * NS
CHUNK = 128
ROWS_PT = NP // NS


def _sc_degree(dst, w):
    e_pad = dst.shape[0]
    epw = e_pad // NW
    nch = epw // CHUNK
    mesh = plsc.VectorSubcoreMesh(core_axis_name="c", subcore_axis_name="s")

    @functools.partial(
        pl.kernel,
        out_type=jax.ShapeDtypeStruct((NC, NP, L), jnp.float32),
        mesh=mesh,
        scratch_types=[
            pltpu.VMEM((CHUNK,), jnp.int32),
            pltpu.VMEM((CHUNK,), jnp.float32),
            pltpu.VMEM((CHUNK, L), jnp.float32),
            pltpu.VMEM_SHARED((NP, L), jnp.float32),
        ],
    )
    def k(dst_hbm, w_hbm, out_hbm, didx, wv, wrows, acc):
        c = lax.axis_index("c")
        s = lax.axis_index("s")
        wid = c * NS + s
        zero = jnp.zeros((L,), jnp.float32)

        def z1(i, carry):
            wrows[i, :] = zero
            return carry

        lax.fori_loop(0, CHUNK, z1, 0)
        for t in range(ROWS_PT // CHUNK):
            pltpu.sync_copy(wrows, acc.at[pl.ds(s * ROWS_PT + t * CHUNK, CHUNK)])
        plsc.subcore_barrier()

        lanes = lax.broadcasted_iota(jnp.int32, (L,), 0)
        zcol = jnp.zeros((L,), jnp.int32)

        def body(ch, carry):
            base = wid * epw + ch * CHUNK
            pltpu.sync_copy(dst_hbm.at[pl.ds(base, CHUNK)], didx)
            pltpu.sync_copy(w_hbm.at[pl.ds(base, CHUNK)], wv)
            for g in range(CHUNK // L):
                wvec = wv[pl.ds(g * L, L)]
                plsc.store_scatter(wrows, [lanes + g * L, zcol], wvec)
            pltpu.sync_copy(wrows, acc.at[didx], add=True)
            return carry

        lax.fori_loop(0, nch, body, 0)
        plsc.subcore_barrier()
        pltpu.sync_copy(acc.at[pl.ds(s * ROWS_PT, ROWS_PT)],
                        out_hbm.at[c, pl.ds(s * ROWS_PT, ROWS_PT)])

    return k(dst, w)


def _sc_spmm(src, dst, w, g):
    e_pad = src.shape[0]
    epw = e_pad // NW
    nch = epw // CHUNK
    mesh = plsc.VectorSubcoreMesh(core_axis_name="c", subcore_axis_name="s")

    @functools.partial(
        pl.kernel,
        out_type=jax.ShapeDtypeStruct((NC, NP, D), jnp.float32),
        mesh=mesh,
        scratch_types=[
            pltpu.VMEM((CHUNK,), jnp.int32),
            pltpu.VMEM((CHUNK,), jnp.int32),
            pltpu.VMEM((CHUNK,), jnp.float32),
            pltpu.VMEM((CHUNK, D), jnp.float32),
            pltpu.VMEM_SHARED((NP, D), jnp.float32),
            pltpu.SemaphoreType.DMA,
        ],
    )
    def k(src_hbm, dst_hbm, w_hbm, g_hbm, out_hbm, sidx, didx, wv, rows, acc, sem):
        c = lax.axis_index("c")
        s = lax.axis_index("s")
        wid = c * NS + s
        zero = jnp.zeros((L,), jnp.float32)

        def z1(i, carry):
            for j in range(D // L):
                rows[i, pl.ds(j * L, L)] = zero
            return carry

        lax.fori_loop(0, CHUNK, z1, 0)
        for t in range(ROWS_PT // CHUNK):
            pltpu.sync_copy(rows, acc.at[pl.ds(s * ROWS_PT + t * CHUNK, CHUNK)])
        plsc.subcore_barrier()

        def body(ch, carry):
            base = wid * epw + ch * CHUNK
            pltpu.sync_copy(src_hbm.at[pl.ds(base, CHUNK)], sidx)
            pltpu.sync_copy(dst_hbm.at[pl.ds(base, CHUNK)], didx)
            pltpu.sync_copy(w_hbm.at[pl.ds(base, CHUNK)], wv)
            pltpu.async_copy(g_hbm.at[sidx], rows, sem).wait()

            def scale(e, icarry):
                web = jnp.full((L,), wv[e], jnp.float32)
                for j in range(D // L):
                    sl = pl.ds(j * L, L)
                    rows[e, sl] = rows[e, sl] * web
                return icarry

            lax.fori_loop(0, CHUNK, scale, 0)
            pltpu.sync_copy(rows, acc.at[didx], add=True)
            return carry

        lax.fori_loop(0, nch, body, 0)
        plsc.subcore_barrier()
        pltpu.sync_copy(acc.at[pl.ds(s * ROWS_PT, ROWS_PT)],
                        out_hbm.at[c, pl.ds(s * ROWS_PT, ROWS_PT)])

    return k(src, dst, w, g)


def _tc_dinv(deg2):
    def k(deg_ref, out_ref):
        deg = jnp.sum(deg_ref[...], axis=(0, 2))[:N] + 1.0
        out_ref[...] = lax.rsqrt(deg)[:, None]

    return pl.pallas_call(
        k, out_shape=jax.ShapeDtypeStruct((N, 1), jnp.float32))(deg2)


def _tc_scale_mm(h, W, dinv):
    def k(h_ref, w_ref, dinv_ref, out_ref):
        hw = lax.dot_general(h_ref[...], w_ref[...],
                             (((1,), (1,)), ((), ())),
                             preferred_element_type=jnp.float32)
        out_ref[...] = dinv_ref[...] * hw

    return pl.pallas_call(
        k, out_shape=jax.ShapeDtypeStruct((N, W.shape[0]), jnp.float32))(
            h, W, dinv)


def _tc_bn(acc, g, dinv, b, gamma, beta):
    def k(acc_ref, g_ref, dinv_ref, b_ref, ga_ref, be_ref, out_ref):
        a = acc_ref[0, :N, :] + acc_ref[1, :N, :] + g_ref[...]
        r = jnp.maximum(dinv_ref[...] * a + b_ref[...], 0.0)
        mu = jnp.mean(r, axis=0, keepdims=True)
        d = r - mu
        var = jnp.mean(d * d, axis=0, keepdims=True)
        out_ref[...] = ga_ref[...] * d * lax.rsqrt(var + 1e-5) + be_ref[...]

    return pl.pallas_call(
        k, out_shape=jax.ShapeDtypeStruct((N, D), jnp.float32))(
            acc, g, dinv, b, gamma, beta)


def _tc_head(bn1, bn2, x, Wih1, bih1, Wih2, bih2, Wl, bl):
    def lstm(gates):
        i = jax.nn.sigmoid(gates[:, :D])
        gg = jnp.tanh(gates[:, 2 * D:3 * D])
        o = jax.nn.sigmoid(gates[:, 3 * D:])
        return o * jnp.tanh(i * gg)

    def k(bn1_ref, bn2_ref, x_ref, wih1_ref, bih1_ref, wih2_ref, bih2_ref,
          wl_ref, bl_ref, out_ref):
        wih1 = wih1_ref[...]
        gates1 = (
            lax.dot_general(bn1_ref[...], wih1[:, :D],
                            (((1,), (1,)), ((), ())),
                            preferred_element_type=jnp.float32)
            + lax.dot_general(bn2_ref[...], wih1[:, D:],
                              (((1,), (1,)), ((), ())),
                              preferred_element_type=jnp.float32)
            + bih1_ref[...])
        h1 = lstm(gates1)
        gates2 = lax.dot_general(h1, wih2_ref[...],
                                 (((1,), (1,)), ((), ())),
                                 preferred_element_type=jnp.float32) + bih2_ref[...]
        h2 = lstm(gates2)
        wl = wl_ref[...]
        y = (lax.dot_general(jnp.maximum(h1, 0.0), wl[:, :D],
                             (((1,), (1,)), ((), ())),
                             preferred_element_type=jnp.float32)
             + lax.dot_general(jnp.maximum(h2, 0.0), wl[:, D:2 * D],
                               (((1,), (1,)), ((), ())),
                               preferred_element_type=jnp.float32)
             + lax.dot_general(jnp.maximum(x_ref[...], 0.0), wl[:, 2 * D:],
                               (((1,), (1,)), ((), ())),
                               preferred_element_type=jnp.float32))
        out_ref[...] = jnp.tanh(y + bl_ref[...])

    return pl.pallas_call(
        k, out_shape=jax.ShapeDtypeStruct((N, 1), jnp.float32))(
            bn1, bn2, x, Wih1, bih1, Wih2, bih2, Wl, bl)


def kernel(x, edge_index, edge_weight, W1, b1, gamma1, beta1, W2, b2,
           gamma2, beta2, Wih1, Whh1, bih1, bhh1, Wih2, Whh2, bih2, bhh2,
           Wl, bl):
    e = edge_index.shape[1]
    block = NW * CHUNK
    e_pad = ((e + block - 1) // block) * block
    pad = e_pad - e
    src = jnp.pad(edge_index[0], (0, pad))
    dst = jnp.pad(edge_index[1], (0, pad))
    w = jnp.pad(edge_weight, (0, pad))

    deg2 = _sc_degree(dst, w)
    dinv = _tc_dinv(deg2)

    g1 = _tc_scale_mm(x, W1, dinv)
    acc1 = _sc_spmm(src, dst, w, g1)
    bn1 = _tc_bn(acc1, g1, dinv, b1[None, :], gamma1[None, :], beta1[None, :])

    g2 = _tc_scale_mm(bn1, W2, dinv)
    acc2 = _sc_spmm(src, dst, w, g2)
    bn2 = _tc_bn(acc2, g2, dinv, b2[None, :], gamma2[None, :], beta2[None, :])

    y = _tc_head(bn1, bn2, x, Wih1, (bih1 + bhh1)[None, :],
                 Wih2, (bih2 + bhh2)[None, :], Wl, (bl + 0.0)[None, :])
    return y


# speedup vs baseline: 7.3133x; 7.3133x over previous
import functools

import jax
import jax.numpy as jnp
from jax import lax
from jax.experimental import pallas as pl
from jax.experimental.pallas import tpu as pltpu
from jax.experimental.pallas import tpu_sc as plsc

N = 10000
NP = 10240
D = 128
NC, NS, L = 2, 16, 16
NW = NC * NS
CHUNK = 128
ROWS_PT = NP // NS


def _sc_degree(dst, w):
    e_pad = dst.shape[0]
    epw = e_pad // NW
    nch = epw // CHUNK
    mesh = plsc.VectorSubcoreMesh(core_axis_name="c", subcore_axis_name="s", num_cores=NC, num_subcores=NS)

    @functools.partial(
        pl.kernel,
        out_type=jax.ShapeDtypeStruct((NC, NP, D), jnp.float32),
        mesh=mesh,
        scratch_types=[
            pltpu.VMEM((CHUNK,), jnp.int32),
            pltpu.VMEM((CHUNK,), jnp.float32),
            pltpu.VMEM((CHUNK, D), jnp.float32),
            pltpu.VMEM_SHARED((NP, D), jnp.float32),
        ],
    )
    def k(dst_hbm, w_hbm, out_hbm, didx, wv, wrows, acc):
        c = lax.axis_index("c")
        s = lax.axis_index("s")
        wid = c * NS + s
        zero = jnp.zeros((L,), jnp.float32)

        def z1(i, carry):
            for j in range(D // L):
                wrows[i, pl.ds(j * L, L)] = zero
            return carry

        lax.fori_loop(0, CHUNK, z1, 0)
        for t in range(ROWS_PT // CHUNK):
            pltpu.sync_copy(wrows, acc.at[pl.ds(s * ROWS_PT + t * CHUNK, CHUNK)])
        plsc.subcore_barrier()

        def body(ch, carry):
            base = wid * epw + ch * CHUNK
            pltpu.sync_copy(dst_hbm.at[pl.ds(base, CHUNK)], didx)
            pltpu.sync_copy(w_hbm.at[pl.ds(base, CHUNK)], wv)

            def put(gidx, icarry):
                w16 = wv[pl.ds(gidx * L, L)]
                for j in range(L):
                    web = jnp.broadcast_to(w16[j], (L,))
                    e = gidx * L + j
                    for kb in range(D // L):
                        wrows[e, pl.ds(kb * L, L)] = web
                return icarry

            lax.fori_loop(0, CHUNK // L, put, 0)
            pltpu.sync_copy(wrows, acc.at[didx], add=True)
            return carry

        lax.fori_loop(0, nch, body, 0)
        plsc.subcore_barrier()
        pltpu.sync_copy(acc.at[pl.ds(s * ROWS_PT, ROWS_PT)],
                        out_hbm.at[c, pl.ds(s * ROWS_PT, ROWS_PT)])

    return k(dst, w)


def _sc_spmm(src, dst, w, g):
    e_pad = src.shape[0]
    epw = e_pad // NW
    nch = epw // CHUNK
    mesh = plsc.VectorSubcoreMesh(core_axis_name="c", subcore_axis_name="s", num_cores=NC, num_subcores=NS)

    @functools.partial(
        pl.kernel,
        out_type=jax.ShapeDtypeStruct((NC, NP, D), jnp.float32),
        mesh=mesh,
        scratch_types=[
            pltpu.VMEM((CHUNK,), jnp.int32),
            pltpu.VMEM((CHUNK,), jnp.int32),
            pltpu.VMEM((CHUNK,), jnp.float32),
            pltpu.VMEM((CHUNK, D), jnp.float32),
            pltpu.VMEM_SHARED((NP, D), jnp.float32),
            pltpu.SemaphoreType.DMA,
        ],
    )
    def k(src_hbm, dst_hbm, w_hbm, g_hbm, out_hbm, sidx, didx, wv, rows, acc, sem):
        c = lax.axis_index("c")
        s = lax.axis_index("s")
        wid = c * NS + s
        zero = jnp.zeros((L,), jnp.float32)

        def z1(i, carry):
            for j in range(D // L):
                rows[i, pl.ds(j * L, L)] = zero
            return carry

        lax.fori_loop(0, CHUNK, z1, 0)
        for t in range(ROWS_PT // CHUNK):
            pltpu.sync_copy(rows, acc.at[pl.ds(s * ROWS_PT + t * CHUNK, CHUNK)])
        plsc.subcore_barrier()

        def body(ch, carry):
            base = wid * epw + ch * CHUNK
            pltpu.sync_copy(src_hbm.at[pl.ds(base, CHUNK)], sidx)
            pltpu.sync_copy(dst_hbm.at[pl.ds(base, CHUNK)], didx)
            pltpu.sync_copy(w_hbm.at[pl.ds(base, CHUNK)], wv)
            pltpu.async_copy(g_hbm.at[sidx], rows, sem).wait()

            def scale(gidx, icarry):
                w16 = wv[pl.ds(gidx * L, L)]
                for j in range(L):
                    web = jnp.broadcast_to(w16[j], (L,))
                    e = gidx * L + j
                    for kb in range(D // L):
                        sl = pl.ds(kb * L, L)
                        rows[e, sl] = rows[e, sl] * web
                return icarry

            lax.fori_loop(0, CHUNK // L, scale, 0)
            pltpu.sync_copy(rows, acc.at[didx], add=True)
            return carry

        lax.fori_loop(0, nch, body, 0)
        plsc.subcore_barrier()
        pltpu.sync_copy(acc.at[pl.ds(s * ROWS_PT, ROWS_PT)],
                        out_hbm.at[c, pl.ds(s * ROWS_PT, ROWS_PT)])

    return k(src, dst, w, g)


def _tc_dinv(deg2):
    def k(deg_ref, out_ref):
        deg = jnp.sum(deg_ref[...], axis=(0, 2))[:N] * (1.0 / D) + 1.0
        out_ref[...] = (1.0 / jnp.sqrt(deg))[:, None]

    return pl.pallas_call(
        k, out_shape=jax.ShapeDtypeStruct((N, 1), jnp.float32))(deg2)


def _tc_scale_mm(h, W, dinv):
    def k(h_ref, w_ref, dinv_ref, out_ref):
        hw = lax.dot_general(h_ref[...], w_ref[...],
                             (((1,), (1,)), ((), ())),
                             preferred_element_type=jnp.float32)
        out_ref[...] = dinv_ref[...] * hw

    return pl.pallas_call(
        k, out_shape=jax.ShapeDtypeStruct((N, W.shape[0]), jnp.float32))(
            h, W, dinv)


def _tc_bn(acc, g, dinv, b, gamma, beta):
    def k(acc_ref, g_ref, dinv_ref, b_ref, ga_ref, be_ref, out_ref):
        a = acc_ref[0, :N, :] + acc_ref[1, :N, :] + g_ref[...]
        r = jnp.maximum(dinv_ref[...] * a + b_ref[...], 0.0)
        mu = jnp.mean(r, axis=0, keepdims=True)
        d = r - mu
        var = jnp.mean(d * d, axis=0, keepdims=True)
        out_ref[...] = ga_ref[...] * d * (1.0 / jnp.sqrt(var + 1e-5)) + be_ref[...]

    return pl.pallas_call(
        k, out_shape=jax.ShapeDtypeStruct((N, D), jnp.float32))(
            acc, g, dinv, b, gamma, beta)


def _tc_head(bn1, bn2, x, Wih1, bih1, Wih2, bih2, Wl, bl):
    def lstm(gates):
        i = jax.nn.sigmoid(gates[:, :D])
        gg = jnp.tanh(gates[:, 2 * D:3 * D])
        o = jax.nn.sigmoid(gates[:, 3 * D:])
        return o * jnp.tanh(i * gg)

    def k(bn1_ref, bn2_ref, x_ref, wih1_ref, bih1_ref, wih2_ref, bih2_ref,
          wl_ref, bl_ref, out_ref):
        wih1 = wih1_ref[...]
        gates1 = (
            lax.dot_general(bn1_ref[...], wih1[:, :D],
                            (((1,), (1,)), ((), ())),
                            preferred_element_type=jnp.float32)
            + lax.dot_general(bn2_ref[...], wih1[:, D:],
                              (((1,), (1,)), ((), ())),
                              preferred_element_type=jnp.float32)
            + bih1_ref[...])
        h1 = lstm(gates1)
        gates2 = lax.dot_general(h1, wih2_ref[...],
                                 (((1,), (1,)), ((), ())),
                                 preferred_element_type=jnp.float32) + bih2_ref[...]
        h2 = lstm(gates2)
        wl = wl_ref[...]
        y = (lax.dot_general(jnp.maximum(h1, 0.0), wl[:, :D],
                             (((1,), (1,)), ((), ())),
                             preferred_element_type=jnp.float32)
             + lax.dot_general(jnp.maximum(h2, 0.0), wl[:, D:2 * D],
                               (((1,), (1,)), ((), ())),
                               preferred_element_type=jnp.float32)
             + lax.dot_general(jnp.maximum(x_ref[...], 0.0), wl[:, 2 * D:],
                               (((1,), (1,)), ((), ())),
                               preferred_element_type=jnp.float32))
        out_ref[...] = jnp.tanh(y + bl_ref[...])

    return pl.pallas_call(
        k, out_shape=jax.ShapeDtypeStruct((N, 1), jnp.float32))(
            bn1, bn2, x, Wih1, bih1, Wih2, bih2, Wl, bl)


def kernel(x, edge_index, edge_weight, W1, b1, gamma1, beta1, W2, b2,
           gamma2, beta2, Wih1, Whh1, bih1, bhh1, Wih2, Whh2, bih2, bhh2,
           Wl, bl):
    e = edge_index.shape[1]
    block = NW * CHUNK
    e_pad = ((e + block - 1) // block) * block
    pad = e_pad - e
    src = jnp.pad(edge_index[0], (0, pad))
    dst = jnp.pad(edge_index[1], (0, pad))
    w = jnp.pad(edge_weight, (0, pad))

    deg2 = _sc_degree(dst, w)
    dinv = _tc_dinv(deg2)

    g1 = _tc_scale_mm(x, W1, dinv)
    acc1 = _sc_spmm(src, dst, w, g1)
    bn1 = _tc_bn(acc1, g1, dinv, b1[None, :], gamma1[None, :], beta1[None, :])

    g2 = _tc_scale_mm(bn1, W2, dinv)
    acc2 = _sc_spmm(src, dst, w, g2)
    bn2 = _tc_bn(acc2, g2, dinv, b2[None, :], gamma2[None, :], beta2[None, :])

    y = _tc_head(bn1, bn2, x, Wih1, (bih1 + bhh1)[None, :],
                 Wih2, (bih2 + bhh2)[None, :], Wl, (bl + 0.0)[None, :])
    return y


# speedup vs baseline: 9.3389x; 1.2770x over previous
import functools

import jax
import jax.numpy as jnp
from jax import lax
from jax.experimental import pallas as pl
from jax.experimental.pallas import tpu as pltpu
from jax.experimental.pallas import tpu_sc as plsc

N = 10000
NP = 10240
D = 128
NC, NS, L = 2, 16, 16
NW = NC * NS
CHUNK = 128
ROWS_PT = NP // NS


def _mesh():
    return plsc.VectorSubcoreMesh(core_axis_name="c", subcore_axis_name="s",
                                  num_cores=NC, num_subcores=NS)


def _sc_degree(dst2, w2):
    nch = dst2.shape[0] // NW

    @functools.partial(
        pl.kernel,
        out_type=jax.ShapeDtypeStruct((NC, NP, D), jnp.float32),
        mesh=_mesh(),
        scratch_types=[
            pltpu.VMEM((4, CHUNK), jnp.int32),
            pltpu.VMEM((4, CHUNK), jnp.float32),
            pltpu.VMEM((CHUNK, D), jnp.float32),
            pltpu.VMEM((CHUNK, D), jnp.float32),
            pltpu.VMEM_SHARED((NP, D), jnp.float32),
            pltpu.SemaphoreType.DMA,
            pltpu.SemaphoreType.DMA,
            pltpu.SemaphoreType.DMA,
            pltpu.SemaphoreType.DMA,
            pltpu.SemaphoreType.DMA,
            pltpu.SemaphoreType.DMA,
        ],
    )
    def k(dst_hbm, w_hbm, out_hbm, didx, wv, w0, w1, acc,
          s0, s1, i0, i1, i2, i3):
        wrows = (w0, w1)
        ssem = (s0, s1)
        isem = (i0, i1, i2, i3)
        c = lax.axis_index("c")
        s = lax.axis_index("s")
        wid = c * NS + s

        def idx_start(ch, slot):
            base = wid * nch + ch
            pltpu.async_copy(dst_hbm.at[pl.ds(base, 1)],
                             didx.at[pl.ds(slot, 1)], isem[slot])
            pltpu.async_copy(w_hbm.at[pl.ds(base, 1)],
                             wv.at[pl.ds(slot, 1)], isem[slot])

        def idx_wait(ch, slot):
            base = wid * nch + ch
            pltpu.make_async_copy(dst_hbm.at[pl.ds(base, 1)],
                                  didx.at[pl.ds(slot, 1)], isem[slot]).wait()
            pltpu.make_async_copy(w_hbm.at[pl.ds(base, 1)],
                                  wv.at[pl.ds(slot, 1)], isem[slot]).wait()

        zero = jnp.zeros((L,), jnp.float32)

        def z1(i, carry):
            for j in range(D // L):
                w0[i, pl.ds(j * L, L)] = zero
            return carry

        lax.fori_loop(0, CHUNK, z1, 0)
        for t in range(ROWS_PT // CHUNK):
            pltpu.sync_copy(w0, acc.at[pl.ds(s * ROWS_PT + t * CHUNK, CHUNK)])
        plsc.subcore_barrier()

        for p in range(2):
            idx_start(p, p)

        def body(i, carry):
            for b in range(4):
                ch = i * 4 + b
                wb, sb = wrows[b % 2], ssem[b % 2]
                wo, so = wrows[1 - b % 2], ssem[1 - b % 2]

                idx_wait(ch, b)

                def put(gidx, icarry):
                    w16 = wv[b, pl.ds(gidx * L, L)]
                    for j in range(L):
                        web = jnp.broadcast_to(w16[j], (L,))
                        e = gidx * L + j
                        for kb in range(D // L):
                            wb[e, pl.ds(kb * L, L)] = web
                    return icarry

                lax.fori_loop(0, CHUNK // L, put, 0)

                @pl.when(ch >= 1)
                def _():
                    pltpu.make_async_copy(
                        wo, acc.at[didx.at[(b + 3) % 4]], so).wait()

                pltpu.async_copy(wb, acc.at[didx.at[b]], sb, add=True)

                @pl.when(ch + 2 < nch)
                def _():
                    idx_start(ch + 2, (b + 2) % 4)
            return carry

        lax.fori_loop(0, nch // 4, body, 0)
        pltpu.make_async_copy(
            wrows[(nch - 1) % 2], acc.at[didx.at[(nch - 1) % 4]],
            ssem[(nch - 1) % 2]).wait()
        plsc.subcore_barrier()
        pltpu.sync_copy(acc.at[pl.ds(s * ROWS_PT, ROWS_PT)],
                        out_hbm.at[c, pl.ds(s * ROWS_PT, ROWS_PT)])

    return k(dst2, w2)


def _sc_spmm(src2, dst2, w2, g):
    nch = src2.shape[0] // NW

    @functools.partial(
        pl.kernel,
        out_type=jax.ShapeDtypeStruct((NC, NP, D), jnp.float32),
        mesh=_mesh(),
        scratch_types=[
            pltpu.VMEM((4, CHUNK), jnp.int32),
            pltpu.VMEM((4, CHUNK), jnp.int32),
            pltpu.VMEM((4, CHUNK), jnp.float32),
            pltpu.VMEM((CHUNK, D), jnp.float32),
            pltpu.VMEM((CHUNK, D), jnp.float32),
            pltpu.VMEM_SHARED((NP, D), jnp.float32),
            pltpu.SemaphoreType.DMA,
            pltpu.SemaphoreType.DMA,
            pltpu.SemaphoreType.DMA,
            pltpu.SemaphoreType.DMA,
            pltpu.SemaphoreType.DMA,
            pltpu.SemaphoreType.DMA,
            pltpu.SemaphoreType.DMA,
            pltpu.SemaphoreType.DMA,
        ],
    )
    def k(src_hbm, dst_hbm, w_hbm, g_hbm, out_hbm, sidx, didx, wv,
          r0, r1, acc, gs0, gs1, ss0, ss1, i0, i1, i2, i3):
        rows = (r0, r1)
        gsem = (gs0, gs1)
        ssem = (ss0, ss1)
        isem = (i0, i1, i2, i3)
        c = lax.axis_index("c")
        s = lax.axis_index("s")
        wid = c * NS + s

        def idx_start(ch, slot):
            base = wid * nch + ch
            pltpu.async_copy(src_hbm.at[pl.ds(base, 1)],
                             sidx.at[pl.ds(slot, 1)], isem[slot])
            pltpu.async_copy(dst_hbm.at[pl.ds(base, 1)],
                             didx.at[pl.ds(slot, 1)], isem[slot])
            pltpu.async_copy(w_hbm.at[pl.ds(base, 1)],
                             wv.at[pl.ds(slot, 1)], isem[slot])

        def idx_wait(ch, slot):
            base = wid * nch + ch
            pltpu.make_async_copy(src_hbm.at[pl.ds(base, 1)],
                                  sidx.at[pl.ds(slot, 1)], isem[slot]).wait()
            pltpu.make_async_copy(dst_hbm.at[pl.ds(base, 1)],
                                  didx.at[pl.ds(slot, 1)], isem[slot]).wait()
            pltpu.make_async_copy(w_hbm.at[pl.ds(base, 1)],
                                  wv.at[pl.ds(slot, 1)], isem[slot]).wait()

        zero = jnp.zeros((L,), jnp.float32)

        def z1(i, carry):
            for j in range(D // L):
                r0[i, pl.ds(j * L, L)] = zero
            return carry

        lax.fori_loop(0, CHUNK, z1, 0)
        for t in range(ROWS_PT // CHUNK):
            pltpu.sync_copy(r0, acc.at[pl.ds(s * ROWS_PT + t * CHUNK, CHUNK)])
        plsc.subcore_barrier()

        idx_start(0, 0)
        idx_start(1, 1)
        idx_wait(0, 0)
        pltpu.async_copy(g_hbm.at[sidx.at[0]], r0, gs0)

        def body(i, carry):
            for b in range(4):
                ch = i * 4 + b
                rb, gb, sb = rows[b % 2], gsem[b % 2], ssem[b % 2]
                ro, go, so = rows[1 - b % 2], gsem[1 - b % 2], ssem[1 - b % 2]
                pltpu.make_async_copy(
                    g_hbm.at[sidx.at[b]], rb, gb).wait()

                @pl.when(ch >= 1)
                def _():
                    pltpu.make_async_copy(
                        ro, acc.at[didx.at[(b + 3) % 4]], so).wait()

                @pl.when(ch + 1 < nch)
                def _():
                    idx_wait(ch + 1, (b + 1) % 4)
                    pltpu.async_copy(g_hbm.at[sidx.at[(b + 1) % 4]], ro, go)

                def scale(gidx, icarry):
                    w16 = wv[b, pl.ds(gidx * L, L)]
                    for j in range(L):
                        web = jnp.broadcast_to(w16[j], (L,))
                        e = gidx * L + j
                        for kb in range(D // L):
                            sl = pl.ds(kb * L, L)
                            rb[e, sl] = rb[e, sl] * web
                    return icarry

                lax.fori_loop(0, CHUNK // L, scale, 0)
                pltpu.async_copy(rb, acc.at[didx.at[b]], sb, add=True)

                @pl.when(ch + 2 < nch)
                def _():
                    idx_start(ch + 2, (b + 2) % 4)
            return carry

        lax.fori_loop(0, nch // 4, body, 0)
        pltpu.make_async_copy(
            rows[(nch - 1) % 2], acc.at[didx.at[(nch - 1) % 4]],
            ssem[(nch - 1) % 2]).wait()
        plsc.subcore_barrier()
        pltpu.sync_copy(acc.at[pl.ds(s * ROWS_PT, ROWS_PT)],
                        out_hbm.at[c, pl.ds(s * ROWS_PT, ROWS_PT)])

    return k(src2, dst2, w2, g)


def _tc_dinv(deg2):
    def k(deg_ref, out_ref):
        deg = jnp.sum(deg_ref[...], axis=(0, 2))[:N] * (1.0 / D) + 1.0
        out_ref[...] = (1.0 / jnp.sqrt(deg))[:, None]

    return pl.pallas_call(
        k, out_shape=jax.ShapeDtypeStruct((N, 1), jnp.float32))(deg2)


def _tc_scale_mm(h, W, dinv):
    def k(h_ref, w_ref, dinv_ref, out_ref):
        hw = lax.dot_general(h_ref[...], w_ref[...],
                             (((1,), (1,)), ((), ())),
                             preferred_element_type=jnp.float32)
        out_ref[...] = dinv_ref[...] * hw

    return pl.pallas_call(
        k, out_shape=jax.ShapeDtypeStruct((N, W.shape[0]), jnp.float32))(
            h, W, dinv)


def _tc_bn(acc, g, dinv, b, gamma, beta):
    def k(acc_ref, g_ref, dinv_ref, b_ref, ga_ref, be_ref, out_ref):
        a = acc_ref[0, :N, :] + acc_ref[1, :N, :] + g_ref[...]
        r = jnp.maximum(dinv_ref[...] * a + b_ref[...], 0.0)
        mu = jnp.mean(r, axis=0, keepdims=True)
        d = r - mu
        var = jnp.mean(d * d, axis=0, keepdims=True)
        out_ref[...] = ga_ref[...] * d * (1.0 / jnp.sqrt(var + 1e-5)) + be_ref[...]

    return pl.pallas_call(
        k, out_shape=jax.ShapeDtypeStruct((N, D), jnp.float32))(
            acc, g, dinv, b, gamma, beta)


def _tc_head(bn1, bn2, x, Wih1, bih1, Wih2, bih2, Wl, bl):
    def lstm(gates):
        i = jax.nn.sigmoid(gates[:, :D])
        gg = jnp.tanh(gates[:, 2 * D:3 * D])
        o = jax.nn.sigmoid(gates[:, 3 * D:])
        return o * jnp.tanh(i * gg)

    def k(bn1_ref, bn2_ref, x_ref, wih1_ref, bih1_ref, wih2_ref, bih2_ref,
          wl_ref, bl_ref, out_ref):
        wih1 = wih1_ref[...]
        gates1 = (
            lax.dot_general(bn1_ref[...], wih1[:, :D],
                            (((1,), (1,)), ((), ())),
                            preferred_element_type=jnp.float32)
            + lax.dot_general(bn2_ref[...], wih1[:, D:],
                              (((1,), (1,)), ((), ())),
                              preferred_element_type=jnp.float32)
            + bih1_ref[...])
        h1 = lstm(gates1)
        gates2 = lax.dot_general(h1, wih2_ref[...],
                                 (((1,), (1,)), ((), ())),
                                 preferred_element_type=jnp.float32) + bih2_ref[...]
        h2 = lstm(gates2)
        wl = wl_ref[...]
        y = (lax.dot_general(jnp.maximum(h1, 0.0), wl[:, :D],
                             (((1,), (1,)), ((), ())),
                             preferred_element_type=jnp.float32)
             + lax.dot_general(jnp.maximum(h2, 0.0), wl[:, D:2 * D],
                               (((1,), (1,)), ((), ())),
                               preferred_element_type=jnp.float32)
             + lax.dot_general(jnp.maximum(x_ref[...], 0.0), wl[:, 2 * D:],
                               (((1,), (1,)), ((), ())),
                               preferred_element_type=jnp.float32))
        out_ref[...] = jnp.tanh(y + bl_ref[...])

    return pl.pallas_call(
        k, out_shape=jax.ShapeDtypeStruct((N, 1), jnp.float32))(
            bn1, bn2, x, Wih1, bih1, Wih2, bih2, Wl, bl)


def kernel(x, edge_index, edge_weight, W1, b1, gamma1, beta1, W2, b2,
           gamma2, beta2, Wih1, Whh1, bih1, bhh1, Wih2, Whh2, bih2, bhh2,
           Wl, bl):
    e = edge_index.shape[1]
    block = NW * CHUNK * 4
    e_pad = ((e + block - 1) // block) * block
    pad = e_pad - e
    src2 = jnp.pad(edge_index[0], (0, pad)).reshape(-1, CHUNK)
    dst2 = jnp.pad(edge_index[1], (0, pad)).reshape(-1, CHUNK)
    w2 = jnp.pad(edge_weight, (0, pad)).reshape(-1, CHUNK)

    deg2 = _sc_degree(dst2, w2)
    dinv = _tc_dinv(deg2)

    g1 = _tc_scale_mm(x, W1, dinv)
    acc1 = _sc_spmm(src2, dst2, w2, g1)
    bn1 = _tc_bn(acc1, g1, dinv, b1[None, :], gamma1[None, :], beta1[None, :])

    g2 = _tc_scale_mm(bn1, W2, dinv)
    acc2 = _sc_spmm(src2, dst2, w2, g2)
    bn2 = _tc_bn(acc2, g2, dinv, b2[None, :], gamma2[None, :], beta2[None, :])

    y = _tc_head(bn1, bn2, x, Wih1, (bih1 + bhh1)[None, :],
                 Wih2, (bih2 + bhh2)[None, :], Wl, (bl + 0.0)[None, :])
    return y
